# Initial kernel scaffold; baseline (speedup 1.0000x reference)
import jax
import jax.numpy as jnp
from jax.experimental import pallas as pl


def kernel(x_cluster, x_fact, memory_bias_cluster, memory_bias_fact, is_clamped_cluster, clamp_hard_cluster, is_clamped_fact, clamp_hard_fact, edge_index_c2f, edge_index_f2c, W_pos_c2f, W_neg_c2f, W_pos_f2c, W_neg_f2c, gate_w_cluster, gate_w_fact, cluster_bias, fact_bias):
    raise NotImplementedError("write your pallas kernel here")


# speedup vs baseline: 4.9108x; 4.9108x over previous
import functools

import jax
import jax.numpy as jnp
from jax import lax
from jax.experimental import pallas as pl
from jax.experimental.pallas import tpu as pltpu
from jax.experimental.pallas import tpu_sc as plsc

_T = 3
_NC, _NF, _E = 10000, 10000, 320000
_DC, _DF = 128, 3
_DFP = 16
_BR = 1000
_GRID = _NC // _BR

_NSC = 2
_NTILES = 16
_NW = _NSC * _NTILES
_EPT = _E // _NW
_CHUNK = 80
_NCHUNK = _EPT // _CHUNK
_NPAD = 10240
_RPT = _NPAD // _NTILES
_ZROWS = 128


def _relu(x):
    return jnp.maximum(x, 0.0)


def _prep_body(xc_r, mbc_r, xf_r, mbf_r, wpc_r, wnc_r, wpf_r, wnf_r,
               lc_o, lf_o, msgf_o, msgc_o):
    lc = xc_r[...] + mbc_r[...]
    lf = xf_r[...] + mbf_r[...]
    lc_o[...] = lc
    lf_o[...] = lf
    msgf_o[...] = (_relu(jnp.dot(lf, wpf_r[...], preferred_element_type=jnp.float32))
                   - _relu(jnp.dot(lf, wnf_r[...], preferred_element_type=jnp.float32)))
    msgc_o[...] = (_relu(jnp.dot(lc, wpc_r[...], preferred_element_type=jnp.float32))
                   - _relu(jnp.dot(lc, wnc_r[...], preferred_element_type=jnp.float32)))


def _step_core(lc_r, lf_r, dcp_r, dfp_r, mc_r, mf_r, gwc_r, gwf_r, bc_r, bf_r):
    lc = lc_r[...]
    lf = lf_r[...]
    dc = dcp_r[0] + dcp_r[1]
    df = dfp_r[0] + dfp_r[1]
    gc = jax.nn.sigmoid(jnp.dot(lc, gwc_r[...], preferred_element_type=jnp.float32))
    gf = jax.nn.sigmoid(jnp.dot(lf, gwf_r[...], preferred_element_type=jnp.float32))
    lcn = lc + (dc * gc + bc_r[...]) * mc_r[...]
    lfn = lf + (df * gf + bf_r[...]) * mf_r[...]
    return lcn, lfn


def _step_next_body(lc_r, lf_r, dcp_r, dfp_r, mc_r, mf_r, gwc_r, gwf_r,
                    bc_r, bf_r, wpc_r, wnc_r, wpf_r, wnf_r,
                    lc_o, lf_o, msgf_o, msgc_o):
    lcn, lfn = _step_core(lc_r, lf_r, dcp_r, dfp_r, mc_r, mf_r,
                          gwc_r, gwf_r, bc_r, bf_r)
    lc_o[...] = lcn
    lf_o[...] = lfn
    msgf_o[...] = (_relu(jnp.dot(lfn, wpf_r[...], preferred_element_type=jnp.float32))
                   - _relu(jnp.dot(lfn, wnf_r[...], preferred_element_type=jnp.float32)))
    msgc_o[...] = (_relu(jnp.dot(lcn, wpc_r[...], preferred_element_type=jnp.float32))
                   - _relu(jnp.dot(lcn, wnc_r[...], preferred_element_type=jnp.float32)))


def _step_last_body(lc_r, lf_r, dcp_r, dfp_r, mc_r, mf_r, gwc_r, gwf_r,
                    bc_r, bf_r, lc_o, lf_o):
    lcn, lfn = _step_core(lc_r, lf_r, dcp_r, dfp_r, mc_r, mf_r,
                          gwc_r, gwf_r, bc_r, bf_r)
    lc_o[...] = lcn
    lf_o[...] = lfn


def _row_spec(d):
    return pl.BlockSpec((_BR, d), lambda i: (i, 0))


def _fixed_spec(shape):
    nd = len(shape)
    return pl.BlockSpec(shape, lambda i, _n=nd: (0,) * _n)


def _part_spec(d):
    return pl.BlockSpec((2, _BR, d), lambda i: (0, i, 0))


def _f32(shape):
    return jax.ShapeDtypeStruct(shape, jnp.float32)


def _build_tc(interpret=False):
    prep = pl.pallas_call(
        _prep_body,
        grid=(_GRID,),
        in_specs=[
            _row_spec(_DC), _row_spec(_DC), _row_spec(_DFP), _row_spec(_DFP),
            _fixed_spec((_DC, _DFP)), _fixed_spec((_DC, _DFP)),
            _fixed_spec((_DFP, _DC)), _fixed_spec((_DFP, _DC)),
        ],
        out_specs=[
            _row_spec(_DC), _row_spec(_DFP), _row_spec(_DC), _row_spec(_DFP),
        ],
        out_shape=[
            _f32((_NC, _DC)), _f32((_NF, _DFP)),
            _f32((_NF, _DC)), _f32((_NC, _DFP)),
        ],
        interpret=interpret,
    )
    common_in = [
        _row_spec(_DC), _row_spec(_DFP), _part_spec(_DC), _part_spec(_DFP),
        _row_spec(1), _row_spec(1),
        _fixed_spec((_DC, 1)), _fixed_spec((_DFP, 1)),
        _fixed_spec((1, _DC)), _fixed_spec((1, _DFP)),
    ]
    step_next = pl.pallas_call(
        _step_next_body,
        grid=(_GRID,),
        in_specs=common_in + [
            _fixed_spec((_DC, _DFP)), _fixed_spec((_DC, _DFP)),
            _fixed_spec((_DFP, _DC)), _fixed_spec((_DFP, _DC)),
        ],
        out_specs=[
            _row_spec(_DC), _row_spec(_DFP), _row_spec(_DC), _row_spec(_DFP),
        ],
        out_shape=[
            _f32((_NC, _DC)), _f32((_NF, _DFP)),
            _f32((_NF, _DC)), _f32((_NC, _DFP)),
        ],
        interpret=interpret,
    )
    step_last = pl.pallas_call(
        _step_last_body,
        grid=(_GRID,),
        in_specs=list(common_in),
        out_specs=[_row_spec(_DC), _row_spec(_DFP)],
        out_shape=[_f32((_NC, _DC)), _f32((_NF, _DFP))],
        interpret=interpret,
    )
    return prep, step_next, step_last


_prep, _step_next, _step_last = _build_tc()


def _sc_body(msgf_hbm, msgc_hbm, sf_hbm, df_hbm, sc_hbm, dc_hbm,
             dcp_out, dfp_out,
             src_v, dst_v, src2_v, dst2_v, rows_c, rows_f, zb, zbf,
             acc_c, acc_f):
    ci = lax.axis_index("c")
    si = lax.axis_index("s")
    gid = ci * _NTILES + si

    zero = jnp.zeros((16,), jnp.float32)

    def zrow(i, carry):
        def zcol(j, c2):
            zb[i, pl.ds(j * 16, 16)] = zero
            return c2
        return lax.fori_loop(0, _DC // 16, zcol, carry)

    lax.fori_loop(0, _ZROWS, zrow, 0)

    def zrowf(i, carry):
        zbf[i, pl.ds(0, 16)] = zero
        return carry

    lax.fori_loop(0, _RPT, zrowf, 0)

    r0 = si * _RPT
    for k in range(_RPT // _ZROWS):
        pltpu.sync_copy(zb, acc_c.at[pl.ds(r0 + k * _ZROWS, _ZROWS)])
    pltpu.sync_copy(zbf, acc_f.at[pl.ds(r0, _RPT)])
    plsc.subcore_barrier()

    base0 = gid * _EPT

    def edge_body(i, carry):
        b = base0 + i * _CHUNK
        pltpu.sync_copy(sf_hbm.at[pl.ds(b, _CHUNK)], src_v)
        pltpu.sync_copy(df_hbm.at[pl.ds(b, _CHUNK)], dst_v)
        pltpu.sync_copy(sc_hbm.at[pl.ds(b, _CHUNK)], src2_v)
        pltpu.sync_copy(dc_hbm.at[pl.ds(b, _CHUNK)], dst2_v)
        pltpu.sync_copy(msgf_hbm.at[src_v], rows_c)
        pltpu.sync_copy(rows_c, acc_c.at[dst_v], add=True)
        pltpu.sync_copy(msgc_hbm.at[src2_v], rows_f)
        pltpu.sync_copy(rows_f, acc_f.at[dst2_v], add=True)
        return carry

    lax.fori_loop(0, _NCHUNK, edge_body, 0)
    plsc.subcore_barrier()

    pltpu.sync_copy(acc_c.at[pl.ds(r0, _RPT)], dcp_out.at[ci, pl.ds(r0, _RPT)])
    pltpu.sync_copy(acc_f.at[pl.ds(r0, _RPT)], dfp_out.at[ci, pl.ds(r0, _RPT)])


@functools.cache
def _get_sc_spmm():
    return pl.kernel(
        _sc_body,
        out_type=(_f32((_NSC, _NPAD, _DC)), _f32((_NSC, _NPAD, _DFP))),
        mesh=plsc.VectorSubcoreMesh(core_axis_name="c", subcore_axis_name="s"),
        compiler_params=pltpu.CompilerParams(use_tc_tiling_on_sc=False),
        scratch_types=[
            pltpu.VMEM((_CHUNK,), jnp.int32),
            pltpu.VMEM((_CHUNK,), jnp.int32),
            pltpu.VMEM((_CHUNK,), jnp.int32),
            pltpu.VMEM((_CHUNK,), jnp.int32),
            pltpu.VMEM((_CHUNK, _DC), jnp.float32),
            pltpu.VMEM((_CHUNK, _DFP), jnp.float32),
            pltpu.VMEM((_ZROWS, _DC), jnp.float32),
            pltpu.VMEM((_RPT, _DFP), jnp.float32),
            pltpu.VMEM_SHARED((_NPAD, _DC), jnp.float32),
            pltpu.VMEM_SHARED((_NPAD, _DFP), jnp.float32),
        ],
    )


def kernel(x_cluster, x_fact, memory_bias_cluster, memory_bias_fact,
           is_clamped_cluster, clamp_hard_cluster, is_clamped_fact,
           clamp_hard_fact, edge_index_c2f, edge_index_f2c,
           W_pos_c2f, W_neg_c2f, W_pos_f2c, W_neg_f2c,
           gate_w_cluster, gate_w_fact, cluster_bias, fact_bias):
    f32 = jnp.float32
    pad = _DFP - _DF
    xf = jnp.pad(x_fact, ((0, 0), (0, pad)))
    mbf = jnp.pad(memory_bias_fact, ((0, 0), (0, pad)))
    wpc_t = jnp.pad(W_pos_c2f.T, ((0, 0), (0, pad)))
    wnc_t = jnp.pad(W_neg_c2f.T, ((0, 0), (0, pad)))
    wpf_t = jnp.pad(W_pos_f2c.T, ((0, pad), (0, 0)))
    wnf_t = jnp.pad(W_neg_f2c.T, ((0, pad), (0, 0)))
    gwf_p = jnp.pad(gate_w_fact, ((0, pad), (0, 0)))
    bc = cluster_bias[None, :]
    bf = jnp.pad(fact_bias, (0, pad))[None, :]
    mask_c = jnp.logical_not(
        is_clamped_cluster & clamp_hard_cluster).astype(f32)[:, None]
    mask_f = jnp.logical_not(
        is_clamped_fact & clamp_hard_fact).astype(f32)[:, None]
    src_f2c = edge_index_f2c[0].astype(jnp.int32)
    dst_f2c = edge_index_f2c[1].astype(jnp.int32)
    src_c2f = edge_index_c2f[0].astype(jnp.int32)
    dst_c2f = edge_index_c2f[1].astype(jnp.int32)

    sc_spmm = _get_sc_spmm()
    lc, lf, msgf, msgc = _prep(x_cluster, memory_bias_cluster, xf, mbf,
                               wpc_t, wnc_t, wpf_t, wnf_t)
    for t in range(_T):
        dcp, dfp = sc_spmm(msgf, msgc, src_f2c, dst_f2c, src_c2f, dst_c2f)
        if t < _T - 1:
            lc, lf, msgf, msgc = _step_next(
                lc, lf, dcp, dfp, mask_c, mask_f, gate_w_cluster, gwf_p,
                bc, bf, wpc_t, wnc_t, wpf_t, wnf_t)
        else:
            lc, lf = _step_last(lc, lf, dcp, dfp, mask_c, mask_f,
                                gate_w_cluster, gwf_p, bc, bf)
    return lc, lf[:, :_DF]


# speedup vs baseline: 11.9182x; 2.4269x over previous
import functools

import jax
import jax.numpy as jnp
from jax import lax
from jax.experimental import pallas as pl
from jax.experimental.pallas import tpu as pltpu
from jax.experimental.pallas import tpu_sc as plsc

_T = 3
_NC, _NF, _E = 10000, 10000, 320000
_DC, _DF = 128, 3
_DFP = 16
_BR = 1000
_GRID = _NC // _BR

_NSC = 2
_NTILES = 16
_NW = _NSC * _NTILES
_EPT = _E // _NW
_CHUNK = 80
_NCHUNK = _EPT // _CHUNK
_NBUF = 3
_NPAD = 10240
_RPT = _NPAD // _NTILES
_ZROWS = 128


def _relu(x):
    return jnp.maximum(x, 0.0)


def _prep_body(xc_r, mbc_r, xf_r, mbf_r, wpc_r, wnc_r, wpf_r, wnf_r,
               lc_o, lf_o, msgf_o, msgc_o):
    lc = xc_r[...] + mbc_r[...]
    lf = xf_r[...] + mbf_r[...]
    lc_o[...] = lc
    lf_o[...] = lf
    msgf_o[...] = (_relu(jnp.dot(lf, wpf_r[...], preferred_element_type=jnp.float32))
                   - _relu(jnp.dot(lf, wnf_r[...], preferred_element_type=jnp.float32)))
    msgc_o[...] = (_relu(jnp.dot(lc, wpc_r[...], preferred_element_type=jnp.float32))
                   - _relu(jnp.dot(lc, wnc_r[...], preferred_element_type=jnp.float32)))


def _step_core(lc_r, lf_r, dcp_r, dfp_r, mc_r, mf_r, gwc_r, gwf_r, bc_r, bf_r):
    lc = lc_r[...]
    lf = lf_r[...]
    dc = dcp_r[0] + dcp_r[1]
    df = dfp_r[0] + dfp_r[1]
    gc = jax.nn.sigmoid(jnp.dot(lc, gwc_r[...], preferred_element_type=jnp.float32))
    gf = jax.nn.sigmoid(jnp.dot(lf, gwf_r[...], preferred_element_type=jnp.float32))
    lcn = lc + (dc * gc + bc_r[...]) * mc_r[...]
    lfn = lf + (df * gf + bf_r[...]) * mf_r[...]
    return lcn, lfn


def _step_next_body(lc_r, lf_r, dcp_r, dfp_r, mc_r, mf_r, gwc_r, gwf_r,
                    bc_r, bf_r, wpc_r, wnc_r, wpf_r, wnf_r,
                    lc_o, lf_o, msgf_o, msgc_o):
    lcn, lfn = _step_core(lc_r, lf_r, dcp_r, dfp_r, mc_r, mf_r,
                          gwc_r, gwf_r, bc_r, bf_r)
    lc_o[...] = lcn
    lf_o[...] = lfn
    msgf_o[...] = (_relu(jnp.dot(lfn, wpf_r[...], preferred_element_type=jnp.float32))
                   - _relu(jnp.dot(lfn, wnf_r[...], preferred_element_type=jnp.float32)))
    msgc_o[...] = (_relu(jnp.dot(lcn, wpc_r[...], preferred_element_type=jnp.float32))
                   - _relu(jnp.dot(lcn, wnc_r[...], preferred_element_type=jnp.float32)))


def _step_last_body(lc_r, lf_r, dcp_r, dfp_r, mc_r, mf_r, gwc_r, gwf_r,
                    bc_r, bf_r, lc_o, lf_o):
    lcn, lfn = _step_core(lc_r, lf_r, dcp_r, dfp_r, mc_r, mf_r,
                          gwc_r, gwf_r, bc_r, bf_r)
    lc_o[...] = lcn
    lf_o[...] = lfn


def _row_spec(d):
    return pl.BlockSpec((_BR, d), lambda i: (i, 0))


def _fixed_spec(shape):
    nd = len(shape)
    return pl.BlockSpec(shape, lambda i, _n=nd: (0,) * _n)


def _part_spec(d):
    return pl.BlockSpec((2, _BR, d), lambda i: (0, i, 0))


def _f32(shape):
    return jax.ShapeDtypeStruct(shape, jnp.float32)


def _build_tc(interpret=False):
    prep = pl.pallas_call(
        _prep_body,
        grid=(_GRID,),
        in_specs=[
            _row_spec(_DC), _row_spec(_DC), _row_spec(_DFP), _row_spec(_DFP),
            _fixed_spec((_DC, _DFP)), _fixed_spec((_DC, _DFP)),
            _fixed_spec((_DFP, _DC)), _fixed_spec((_DFP, _DC)),
        ],
        out_specs=[
            _row_spec(_DC), _row_spec(_DFP), _row_spec(_DC), _row_spec(_DFP),
        ],
        out_shape=[
            _f32((_NC, _DC)), _f32((_NF, _DFP)),
            _f32((_NF, _DC)), _f32((_NC, _DFP)),
        ],
        interpret=interpret,
    )
    common_in = [
        _row_spec(_DC), _row_spec(_DFP), _part_spec(_DC), _part_spec(_DFP),
        _row_spec(1), _row_spec(1),
        _fixed_spec((_DC, 1)), _fixed_spec((_DFP, 1)),
        _fixed_spec((1, _DC)), _fixed_spec((1, _DFP)),
    ]
    step_next = pl.pallas_call(
        _step_next_body,
        grid=(_GRID,),
        in_specs=common_in + [
            _fixed_spec((_DC, _DFP)), _fixed_spec((_DC, _DFP)),
            _fixed_spec((_DFP, _DC)), _fixed_spec((_DFP, _DC)),
        ],
        out_specs=[
            _row_spec(_DC), _row_spec(_DFP), _row_spec(_DC), _row_spec(_DFP),
        ],
        out_shape=[
            _f32((_NC, _DC)), _f32((_NF, _DFP)),
            _f32((_NF, _DC)), _f32((_NC, _DFP)),
        ],
        interpret=interpret,
    )
    step_last = pl.pallas_call(
        _step_last_body,
        grid=(_GRID,),
        in_specs=list(common_in),
        out_specs=[_row_spec(_DC), _row_spec(_DFP)],
        out_shape=[_f32((_NC, _DC)), _f32((_NF, _DFP))],
        interpret=interpret,
    )
    return prep, step_next, step_last


_prep, _step_next, _step_last = _build_tc()


def _sc_body(msgf_hbm, msgc_hbm, idx_hbm,
             dcp_out, dfp_out,
             rc0, rc1, rc2, rf0, rf1, rf2, ib0, ib1, ib2,
             acc_c, acc_f,
             gc0, gc1, gc2, gf0, gf1, gf2,
             sc0, sc1, sc2, sf0, sf1, sf2,
             ic0, ic1, ic2):
    ci = lax.axis_index("c")
    si = lax.axis_index("s")
    gid = ci * _NTILES + si

    rcs = (rc0, rc1, rc2)
    rfs = (rf0, rf1, rf2)
    ibs = (ib0, ib1, ib2)
    gcs = (gc0, gc1, gc2)
    gfs = (gf0, gf1, gf2)
    scs = (sc0, sc1, sc2)
    sfs = (sf0, sf1, sf2)
    ics = (ic0, ic1, ic2)

    zero = jnp.zeros((16,), jnp.float32)

    def zrow(i, carry):
        def zcol(j, c2):
            rc0[i, pl.ds(j * 16, 16)] = zero
            return c2
        lax.fori_loop(0, _DC // 16, zcol, carry)
        rf0[i, pl.ds(0, 16)] = zero
        return carry

    lax.fori_loop(0, _CHUNK, zrow, 0)

    r0 = si * _RPT
    for k in range(_RPT // _CHUNK):
        pltpu.sync_copy(rc0, acc_c.at[pl.ds(r0 + k * _CHUNK, _CHUNK)])
        pltpu.sync_copy(rf0, acc_f.at[pl.ds(r0 + k * _CHUNK, _CHUNK)])
    plsc.subcore_barrier()

    def i_issue(j, b):
        pltpu.async_copy(idx_hbm.at[gid, j], ibs[b], ics[b])

    def i_wait(b):
        pltpu.make_async_copy(idx_hbm.at[gid, 0], ibs[b], ics[b]).wait()

    def g_issue(b):
        pltpu.async_copy(msgf_hbm.at[ibs[b].at[0]], rcs[b], gcs[b])
        pltpu.async_copy(msgc_hbm.at[ibs[b].at[2]], rfs[b], gfs[b])

    def g_wait(b):
        pltpu.make_async_copy(msgf_hbm.at[ibs[b].at[0]], rcs[b], gcs[b]).wait()
        pltpu.make_async_copy(msgc_hbm.at[ibs[b].at[2]], rfs[b], gfs[b]).wait()

    def s_issue(b):
        pltpu.async_copy(rcs[b], acc_c.at[ibs[b].at[1]], scs[b], add=True)
        pltpu.async_copy(rfs[b], acc_f.at[ibs[b].at[3]], sfs[b], add=True)

    def s_wait(b):
        pltpu.make_async_copy(rcs[b], acc_c.at[ibs[b].at[1]], scs[b]).wait()
        pltpu.make_async_copy(rfs[b], acc_f.at[ibs[b].at[3]], sfs[b]).wait()

    def ring_body(i, carry):
        for b in range(_NBUF):
            j = i * _NBUF + b
            b2 = (b + 1) % _NBUF
            b1 = (b + 2) % _NBUF

            @pl.when(jnp.logical_and(j >= 3, j < _NCHUNK + 3))
            def _():
                s_wait(b)

            @pl.when(j < _NCHUNK)
            def _():
                i_issue(j, b)

            @pl.when(jnp.logical_and(j >= 2, j < _NCHUNK + 2))
            def _():
                g_wait(b2)
                s_issue(b2)

            @pl.when(jnp.logical_and(j >= 1, j < _NCHUNK + 1))
            def _():
                i_wait(b1)
                g_issue(b1)

        return carry

    lax.fori_loop(0, (_NCHUNK + 3 + _NBUF - 1) // _NBUF, ring_body, 0)
    plsc.subcore_barrier()

    pltpu.sync_copy(acc_c.at[pl.ds(r0, _RPT)], dcp_out.at[ci, pl.ds(r0, _RPT)])
    pltpu.sync_copy(acc_f.at[pl.ds(r0, _RPT)], dfp_out.at[ci, pl.ds(r0, _RPT)])


@functools.cache
def _get_sc_spmm():
    return pl.kernel(
        _sc_body,
        out_type=(_f32((_NSC, _NPAD, _DC)), _f32((_NSC, _NPAD, _DFP))),
        mesh=plsc.VectorSubcoreMesh(core_axis_name="c", subcore_axis_name="s"),
        compiler_params=pltpu.CompilerParams(use_tc_tiling_on_sc=False),
        scratch_types=(
            [pltpu.VMEM((_CHUNK, _DC), jnp.float32)] * _NBUF
            + [pltpu.VMEM((_CHUNK, _DFP), jnp.float32)] * _NBUF
            + [pltpu.VMEM((4, _CHUNK), jnp.int32)] * _NBUF
            + [pltpu.VMEM_SHARED((_NPAD, _DC), jnp.float32),
               pltpu.VMEM_SHARED((_NPAD, _DFP), jnp.float32)]
            + [pltpu.SemaphoreType.DMA] * (5 * _NBUF)
        ),
    )


def kernel(x_cluster, x_fact, memory_bias_cluster, memory_bias_fact,
           is_clamped_cluster, clamp_hard_cluster, is_clamped_fact,
           clamp_hard_fact, edge_index_c2f, edge_index_f2c,
           W_pos_c2f, W_neg_c2f, W_pos_f2c, W_neg_f2c,
           gate_w_cluster, gate_w_fact, cluster_bias, fact_bias):
    f32 = jnp.float32
    pad = _DFP - _DF
    xf = jnp.pad(x_fact, ((0, 0), (0, pad)))
    mbf = jnp.pad(memory_bias_fact, ((0, 0), (0, pad)))
    wpc_t = jnp.pad(W_pos_c2f.T, ((0, 0), (0, pad)))
    wnc_t = jnp.pad(W_neg_c2f.T, ((0, 0), (0, pad)))
    wpf_t = jnp.pad(W_pos_f2c.T, ((0, pad), (0, 0)))
    wnf_t = jnp.pad(W_neg_f2c.T, ((0, pad), (0, 0)))
    gwf_p = jnp.pad(gate_w_fact, ((0, pad), (0, 0)))
    bc = cluster_bias[None, :]
    bf = jnp.pad(fact_bias, (0, pad))[None, :]
    mask_c = jnp.logical_not(
        is_clamped_cluster & clamp_hard_cluster).astype(f32)[:, None]
    mask_f = jnp.logical_not(
        is_clamped_fact & clamp_hard_fact).astype(f32)[:, None]
    eshape = (_NW, _NCHUNK, _CHUNK)
    idx_all = jnp.stack(
        [edge_index_f2c[0].astype(jnp.int32).reshape(eshape),
         edge_index_f2c[1].astype(jnp.int32).reshape(eshape),
         edge_index_c2f[0].astype(jnp.int32).reshape(eshape),
         edge_index_c2f[1].astype(jnp.int32).reshape(eshape)], axis=2)

    sc_spmm = _get_sc_spmm()
    lc, lf, msgf, msgc = _prep(x_cluster, memory_bias_cluster, xf, mbf,
                               wpc_t, wnc_t, wpf_t, wnf_t)
    for t in range(_T):
        dcp, dfp = sc_spmm(msgf, msgc, idx_all)
        if t < _T - 1:
            lc, lf, msgf, msgc = _step_next(
                lc, lf, dcp, dfp, mask_c, mask_f, gate_w_cluster, gwf_p,
                bc, bf, wpc_t, wnc_t, wpf_t, wnf_t)
        else:
            lc, lf = _step_last(lc, lf, dcp, dfp, mask_c, mask_f,
                                gate_w_cluster, gwf_p, bc, bf)
    return lc, lf[:, :_DF]


# speedup vs baseline: 13.8122x; 1.1589x over previous
import functools

import jax
import jax.numpy as jnp
from jax import lax
from jax.experimental import pallas as pl
from jax.experimental.pallas import tpu as pltpu
from jax.experimental.pallas import tpu_sc as plsc

_T = 3
_NC, _NF, _E = 10000, 10000, 320000
_DC, _DF = 128, 3
_DFP = 16
_BR = 1000
_GRID = _NC // _BR

_NSC = 2
_NTILES = 16
_NW = _NSC * _NTILES
_EPT = _E // _NW
_CHUNK = 125
_NCHUNK = _EPT // _CHUNK
_NBUF = 2
_NIB = 3
_STEP_UNROLL = 6
_NPAD = 10240
_RPT = _NPAD // _NTILES
_ZROWS = 128


def _relu(x):
    return jnp.maximum(x, 0.0)


def _prep_body(xc_r, mbc_r, xf_r, mbf_r, wpc_r, wnc_r, wpf_r, wnf_r,
               lc_o, lf_o, msgf_o, msgc_o):
    lc = xc_r[...] + mbc_r[...]
    lf = xf_r[...] + mbf_r[...]
    lc_o[...] = lc
    lf_o[...] = lf
    msgf_o[...] = (_relu(jnp.dot(lf, wpf_r[...], preferred_element_type=jnp.float32))
                   - _relu(jnp.dot(lf, wnf_r[...], preferred_element_type=jnp.float32)))
    msgc_o[...] = (_relu(jnp.dot(lc, wpc_r[...], preferred_element_type=jnp.float32))
                   - _relu(jnp.dot(lc, wnc_r[...], preferred_element_type=jnp.float32)))


def _step_core(lc_r, lf_r, dcp_r, dfp_r, mc_r, mf_r, gwc_r, gwf_r, bc_r, bf_r):
    lc = lc_r[...]
    lf = lf_r[...]
    dc = dcp_r[0] + dcp_r[1]
    df = dfp_r[0] + dfp_r[1]
    gc = jax.nn.sigmoid(jnp.dot(lc, gwc_r[...], preferred_element_type=jnp.float32))
    gf = jax.nn.sigmoid(jnp.dot(lf, gwf_r[...], preferred_element_type=jnp.float32))
    lcn = lc + (dc * gc + bc_r[...]) * mc_r[...]
    lfn = lf + (df * gf + bf_r[...]) * mf_r[...]
    return lcn, lfn


def _step_next_body(lc_r, lf_r, dcp_r, dfp_r, mc_r, mf_r, gwc_r, gwf_r,
                    bc_r, bf_r, wpc_r, wnc_r, wpf_r, wnf_r,
                    lc_o, lf_o, msgf_o, msgc_o):
    lcn, lfn = _step_core(lc_r, lf_r, dcp_r, dfp_r, mc_r, mf_r,
                          gwc_r, gwf_r, bc_r, bf_r)
    lc_o[...] = lcn
    lf_o[...] = lfn
    msgf_o[...] = (_relu(jnp.dot(lfn, wpf_r[...], preferred_element_type=jnp.float32))
                   - _relu(jnp.dot(lfn, wnf_r[...], preferred_element_type=jnp.float32)))
    msgc_o[...] = (_relu(jnp.dot(lcn, wpc_r[...], preferred_element_type=jnp.float32))
                   - _relu(jnp.dot(lcn, wnc_r[...], preferred_element_type=jnp.float32)))


def _step_last_body(lc_r, lf_r, dcp_r, dfp_r, mc_r, mf_r, gwc_r, gwf_r,
                    bc_r, bf_r, lc_o, lf_o):
    lcn, lfn = _step_core(lc_r, lf_r, dcp_r, dfp_r, mc_r, mf_r,
                          gwc_r, gwf_r, bc_r, bf_r)
    lc_o[...] = lcn
    lf_o[...] = lfn


def _row_spec(d):
    return pl.BlockSpec((_BR, d), lambda i: (i, 0))


def _fixed_spec(shape):
    nd = len(shape)
    return pl.BlockSpec(shape, lambda i, _n=nd: (0,) * _n)


def _part_spec(d):
    return pl.BlockSpec((2, _BR, d), lambda i: (0, i, 0))


def _f32(shape):
    return jax.ShapeDtypeStruct(shape, jnp.float32)


def _build_tc(interpret=False):
    prep = pl.pallas_call(
        _prep_body,
        grid=(_GRID,),
        in_specs=[
            _row_spec(_DC), _row_spec(_DC), _row_spec(_DFP), _row_spec(_DFP),
            _fixed_spec((_DC, _DFP)), _fixed_spec((_DC, _DFP)),
            _fixed_spec((_DFP, _DC)), _fixed_spec((_DFP, _DC)),
        ],
        out_specs=[
            _row_spec(_DC), _row_spec(_DFP), _row_spec(_DC), _row_spec(_DFP),
        ],
        out_shape=[
            _f32((_NC, _DC)), _f32((_NF, _DFP)),
            _f32((_NF, _DC)), _f32((_NC, _DFP)),
        ],
        interpret=interpret,
    )
    common_in = [
        _row_spec(_DC), _row_spec(_DFP), _part_spec(_DC), _part_spec(_DFP),
        _row_spec(1), _row_spec(1),
        _fixed_spec((_DC, 1)), _fixed_spec((_DFP, 1)),
        _fixed_spec((1, _DC)), _fixed_spec((1, _DFP)),
    ]
    step_next = pl.pallas_call(
        _step_next_body,
        grid=(_GRID,),
        in_specs=common_in + [
            _fixed_spec((_DC, _DFP)), _fixed_spec((_DC, _DFP)),
            _fixed_spec((_DFP, _DC)), _fixed_spec((_DFP, _DC)),
        ],
        out_specs=[
            _row_spec(_DC), _row_spec(_DFP), _row_spec(_DC), _row_spec(_DFP),
        ],
        out_shape=[
            _f32((_NC, _DC)), _f32((_NF, _DFP)),
            _f32((_NF, _DC)), _f32((_NC, _DFP)),
        ],
        interpret=interpret,
    )
    step_last = pl.pallas_call(
        _step_last_body,
        grid=(_GRID,),
        in_specs=list(common_in),
        out_specs=[_row_spec(_DC), _row_spec(_DFP)],
        out_shape=[_f32((_NC, _DC)), _f32((_NF, _DFP))],
        interpret=interpret,
    )
    return prep, step_next, step_last


_prep, _step_next, _step_last = _build_tc()


def _sc_body(msgf_hbm, msgc_hbm, idx_hbm,
             dcp_out, dfp_out,
             rc0, rc1, rf0, rf1, ib0, ib1, ib2,
             acc_c, acc_f,
             gc0, gc1, gf0, gf1,
             sc0, sc1, sf0, sf1,
             ic0, ic1, ic2):
    ci = lax.axis_index("c")
    si = lax.axis_index("s")
    gid = ci * _NTILES + si

    rcs = (rc0, rc1)
    rfs = (rf0, rf1)
    ibs = (ib0, ib1, ib2)
    gcs = (gc0, gc1)
    gfs = (gf0, gf1)
    scs = (sc0, sc1)
    sfs = (sf0, sf1)
    ics = (ic0, ic1, ic2)

    zero = jnp.zeros((16,), jnp.float32)

    def zrow(i, carry):
        def zcol(j, c2):
            rc0[i, pl.ds(j * 16, 16)] = zero
            return c2
        lax.fori_loop(0, _DC // 16, zcol, carry)
        rf0[i, pl.ds(0, 16)] = zero
        return carry

    _ZR = 80
    lax.fori_loop(0, _ZR, zrow, 0)

    r0 = si * _RPT
    for k in range(_RPT // _ZR):
        pltpu.sync_copy(rc0.at[pl.ds(0, _ZR)],
                        acc_c.at[pl.ds(r0 + k * _ZR, _ZR)])
        pltpu.sync_copy(rf0.at[pl.ds(0, _ZR)],
                        acc_f.at[pl.ds(r0 + k * _ZR, _ZR)])
    plsc.subcore_barrier()

    def i_issue(j, b):
        pltpu.async_copy(idx_hbm.at[gid, j], ibs[b], ics[b])

    def i_wait(b):
        pltpu.make_async_copy(idx_hbm.at[gid, 0], ibs[b], ics[b]).wait()

    def g_issue(b, x):
        pltpu.async_copy(msgf_hbm.at[ibs[x].at[0]], rcs[b], gcs[b])
        pltpu.async_copy(msgc_hbm.at[ibs[x].at[2]], rfs[b], gfs[b])

    def g_wait(b):
        pltpu.make_async_copy(msgf_hbm.at[ibs[0].at[0]], rcs[b], gcs[b]).wait()
        pltpu.make_async_copy(msgc_hbm.at[ibs[0].at[2]], rfs[b], gfs[b]).wait()

    def s_issue(b, x):
        pltpu.async_copy(rcs[b], acc_c.at[ibs[x].at[1]], scs[b], add=True)
        pltpu.async_copy(rfs[b], acc_f.at[ibs[x].at[3]], sfs[b], add=True)

    def s_wait(b):
        pltpu.make_async_copy(rcs[b], acc_c.at[ibs[0].at[1]], scs[b]).wait()
        pltpu.make_async_copy(rfs[b], acc_f.at[ibs[0].at[3]], sfs[b]).wait()

    def ring_body(i, carry):
        for u in range(_STEP_UNROLL):
            j = i * _STEP_UNROLL + u
            db = u % _NBUF
            db1 = (u + 1) % _NBUF
            xb = u % _NIB
            xb1 = (u + 2) % _NIB
            xb2 = (u + 1) % _NIB

            @pl.when(jnp.logical_and(j >= 3, j < _NCHUNK + 3))
            def _():
                s_wait(db1)

            @pl.when(j < _NCHUNK)
            def _():
                i_issue(j, xb)

            @pl.when(jnp.logical_and(j >= 2, j < _NCHUNK + 2))
            def _(db=db, xb2=xb2):
                g_wait(db)
                s_issue(db, xb2)

            @pl.when(jnp.logical_and(j >= 1, j < _NCHUNK + 1))
            def _(db1=db1, xb1=xb1):
                i_wait(xb1)
                g_issue(db1, xb1)

        return carry

    lax.fori_loop(0, (_NCHUNK + 3 + _STEP_UNROLL - 1) // _STEP_UNROLL,
                  ring_body, 0)
    plsc.subcore_barrier()

    pltpu.sync_copy(acc_c.at[pl.ds(r0, _RPT)], dcp_out.at[ci, pl.ds(r0, _RPT)])
    pltpu.sync_copy(acc_f.at[pl.ds(r0, _RPT)], dfp_out.at[ci, pl.ds(r0, _RPT)])


@functools.cache
def _get_sc_spmm():
    return pl.kernel(
        _sc_body,
        out_type=(_f32((_NSC, _NPAD, _DC)), _f32((_NSC, _NPAD, _DFP))),
        mesh=plsc.VectorSubcoreMesh(core_axis_name="c", subcore_axis_name="s"),
        compiler_params=pltpu.CompilerParams(use_tc_tiling_on_sc=False),
        scratch_types=(
            [pltpu.VMEM((_CHUNK, _DC), jnp.float32)] * _NBUF
            + [pltpu.VMEM((_CHUNK, _DFP), jnp.float32)] * _NBUF
            + [pltpu.VMEM((4, _CHUNK), jnp.int32)] * _NIB
            + [pltpu.VMEM_SHARED((_NPAD, _DC), jnp.float32),
               pltpu.VMEM_SHARED((_NPAD, _DFP), jnp.float32)]
            + [pltpu.SemaphoreType.DMA] * (4 * _NBUF + _NIB)
        ),
    )


def kernel(x_cluster, x_fact, memory_bias_cluster, memory_bias_fact,
           is_clamped_cluster, clamp_hard_cluster, is_clamped_fact,
           clamp_hard_fact, edge_index_c2f, edge_index_f2c,
           W_pos_c2f, W_neg_c2f, W_pos_f2c, W_neg_f2c,
           gate_w_cluster, gate_w_fact, cluster_bias, fact_bias):
    f32 = jnp.float32
    pad = _DFP - _DF
    xf = jnp.pad(x_fact, ((0, 0), (0, pad)))
    mbf = jnp.pad(memory_bias_fact, ((0, 0), (0, pad)))
    wpc_t = jnp.pad(W_pos_c2f.T, ((0, 0), (0, pad)))
    wnc_t = jnp.pad(W_neg_c2f.T, ((0, 0), (0, pad)))
    wpf_t = jnp.pad(W_pos_f2c.T, ((0, pad), (0, 0)))
    wnf_t = jnp.pad(W_neg_f2c.T, ((0, pad), (0, 0)))
    gwf_p = jnp.pad(gate_w_fact, ((0, pad), (0, 0)))
    bc = cluster_bias[None, :]
    bf = jnp.pad(fact_bias, (0, pad))[None, :]
    mask_c = jnp.logical_not(
        is_clamped_cluster & clamp_hard_cluster).astype(f32)[:, None]
    mask_f = jnp.logical_not(
        is_clamped_fact & clamp_hard_fact).astype(f32)[:, None]
    eshape = (_NW, _NCHUNK, _CHUNK)
    idx_all = jnp.stack(
        [edge_index_f2c[0].astype(jnp.int32).reshape(eshape),
         edge_index_f2c[1].astype(jnp.int32).reshape(eshape),
         edge_index_c2f[0].astype(jnp.int32).reshape(eshape),
         edge_index_c2f[1].astype(jnp.int32).reshape(eshape)], axis=2)

    sc_spmm = _get_sc_spmm()
    lc, lf, msgf, msgc = _prep(x_cluster, memory_bias_cluster, xf, mbf,
                               wpc_t, wnc_t, wpf_t, wnf_t)
    for t in range(_T):
        dcp, dfp = sc_spmm(msgf, msgc, idx_all)
        if t < _T - 1:
            lc, lf, msgf, msgc = _step_next(
                lc, lf, dcp, dfp, mask_c, mask_f, gate_w_cluster, gwf_p,
                bc, bf, wpc_t, wnc_t, wpf_t, wnf_t)
        else:
            lc, lf = _step_last(lc, lf, dcp, dfp, mask_c, mask_f,
                                gate_w_cluster, gwf_p, bc, bf)
    return lc, lf[:, :_DF]
